# baseline (speedup 1.0000x reference)
import functools
import math

import jax
import jax.numpy as jnp
from jax import lax
from jax.experimental import pallas as pl
from jax.experimental.pallas import tpu as pltpu
from jax.experimental.pallas import tpu_sc as plsc

D = 128
B = 4096
L = 200
BL = B * L
VOCAB = 100000

NC = 2
NS = 16
NW = NC * NS
PER_W = BL // NW
CHUNK = 128
NCHUNK = PER_W // CHUNK
NBUF = 4

_SCALE = math.sqrt(float(D))
_HI_MASK = -65536

_MESH = plsc.VectorSubcoreMesh(core_axis_name="c", subcore_axis_name="s")


@functools.partial(
    pl.kernel,
    mesh=_MESH,
    compiler_params=pltpu.CompilerParams(use_tc_tiling_on_sc=False),
    out_type=jax.ShapeDtypeStruct((BL, D), jnp.float32),
    scratch_types=[
        pltpu.VMEM((NCHUNK, CHUNK), jnp.int32),
        pltpu.VMEM((NBUF, CHUNK, D // 2), jnp.int32),
        pltpu.VMEM((NBUF, CHUNK, D), jnp.float32),
    ]
    + [pltpu.SemaphoreType.DMA] * (2 * NBUF),
)
def _gather_kernel(idx_hbm, table_hbm, out_hbm, idx_v, rows_i, rows_f, *sems):
    gsems = sems[:NBUF]
    osems = sems[NBUF:]
    wid = lax.axis_index("s") * NC + lax.axis_index("c")
    base = wid * PER_W

    pltpu.sync_copy(idx_hbm.at[pl.ds(wid * NCHUNK, NCHUNK)], idx_v)

    lane = lax.iota(jnp.int32, 16)
    parity_odd = (lane & 1) == 1
    perm_half = [(lane >> 1) + 8 * h for h in range(2)]

    def _vreg_gather(x, idx):
        return lax.gather(
            x,
            idx[:, None],
            dimension_numbers=lax.GatherDimensionNumbers(
                offset_dims=(), collapsed_slice_dims=(0,), start_index_map=(0,)
            ),
            slice_sizes=(1,),
            mode=lax.GatherScatterMode.PROMISE_IN_BOUNDS,
        )

    def gather_start(j, b):
        pltpu.async_copy(table_hbm.at[idx_v.at[j]], rows_i.at[b], gsems[b])

    def gather_wait(b):
        pltpu.make_async_copy(
            table_hbm.at[pl.ds(0, CHUNK)], rows_i.at[b], gsems[b]
        ).wait()

    def out_start(j, b):
        pltpu.async_copy(
            rows_f.at[b], out_hbm.at[pl.ds(base + j * CHUNK, CHUNK)], osems[b]
        )

    def out_wait(b):
        pltpu.make_async_copy(
            out_hbm.at[pl.ds(0, CHUNK)], rows_f.at[b], osems[b]
        ).wait()

    def expand_buf(b):
        def row_body(r, carry):
            for j in range(D // 32):
                w = rows_i[b, r, pl.ds(16 * j, 16)]
                for h in range(2):
                    wp = _vreg_gather(w, perm_half[h])
                    bits = jnp.where(parity_odd, wp & _HI_MASK, wp << 16)
                    v = lax.bitcast_convert_type(bits, jnp.float32) * _SCALE
                    rows_f[b, r, pl.ds(32 * j + 16 * h, 16)] = v
            return carry

        lax.fori_loop(0, CHUNK, row_body, 0)

    for b in range(NBUF):
        gather_start(b, b)

    nsteps = NCHUNK // NBUF

    def body(s, carry):
        for b in range(NBUF):
            j = s * NBUF + b
            gather_wait(b)
            expand_buf(b)
            out_start(j, b)

            @pl.when(s < nsteps - 1)
            def _():
                out_wait(b)
                gather_start(j + NBUF, b)

        return carry

    lax.fori_loop(0, nsteps, body, 0)

    for b in range(NBUF):
        out_wait(b)


def kernel(input_ids, word_em):
    idx = input_ids.reshape(BL).astype(jnp.int32).reshape(BL // CHUNK, CHUNK)
    packed = jax.lax.bitcast_convert_type(
        word_em.astype(jnp.bfloat16).reshape(VOCAB, D // 2, 2), jnp.int32
    )
    out = _gather_kernel(idx, packed)
    return out.reshape(B, L, D)


# speedup vs baseline: 4.3661x; 4.3661x over previous
import functools
import math

import jax
import jax.numpy as jnp
from jax import lax
from jax.experimental import pallas as pl
from jax.experimental.pallas import tpu as pltpu
from jax.experimental.pallas import tpu_sc as plsc

D = 128
B = 4096
L = 200
BL = B * L

NC = 2
NS = 16
NW = NC * NS
PER_W = BL // NW
CHUNK = 128
NCHUNK = PER_W // CHUNK
NBUF = 4

_SCALE = math.sqrt(float(D))

_MESH = plsc.VectorSubcoreMesh(core_axis_name="c", subcore_axis_name="s")


@functools.partial(
    pl.kernel,
    mesh=_MESH,
    out_type=jax.ShapeDtypeStruct((BL, D), jnp.float32),
    scratch_types=[
        pltpu.VMEM((NCHUNK, CHUNK), jnp.int32),
        pltpu.VMEM((NBUF, CHUNK, D), jnp.float32),
    ]
    + [pltpu.SemaphoreType.DMA] * (2 * NBUF),
)
def _gather_kernel(idx_hbm, table_hbm, out_hbm, idx_v, rows_v, *sems):
    gsems = sems[:NBUF]
    osems = sems[NBUF:]
    wid = lax.axis_index("s") * NC + lax.axis_index("c")
    base = wid * PER_W

    pltpu.sync_copy(idx_hbm.at[pl.ds(wid * NCHUNK, NCHUNK)], idx_v)

    def gather_start(j, b):
        pltpu.async_copy(table_hbm.at[idx_v.at[j]], rows_v.at[b], gsems[b])

    def gather_wait(b):
        pltpu.make_async_copy(
            table_hbm.at[pl.ds(0, CHUNK)], rows_v.at[b], gsems[b]
        ).wait()

    def out_start(j, b):
        pltpu.async_copy(
            rows_v.at[b], out_hbm.at[pl.ds(base + j * CHUNK, CHUNK)], osems[b]
        )

    def out_wait(b):
        pltpu.make_async_copy(
            out_hbm.at[pl.ds(0, CHUNK)], rows_v.at[b], osems[b]
        ).wait()

    def scale_buf(b):
        def row_body(r, carry):
            for k in range(D // 16):
                sl = pl.ds(k * 16, 16)
                rows_v[b, r, sl] = rows_v[b, r, sl] * _SCALE
            return carry

        lax.fori_loop(0, CHUNK, row_body, 0)

    for b in range(NBUF):
        gather_start(b, b)

    nsteps = NCHUNK // NBUF

    def body(s, carry):
        for b in range(NBUF):
            j = s * NBUF + b
            gather_wait(b)
            scale_buf(b)
            out_start(j, b)

            @pl.when(s < nsteps - 1)
            def _():
                out_wait(b)
                gather_start(j + NBUF, b)

        return carry

    lax.fori_loop(0, nsteps, body, 0)

    for b in range(NBUF):
        out_wait(b)


def kernel(input_ids, word_em):
    idx = input_ids.reshape(BL).astype(jnp.int32).reshape(BL // CHUNK, CHUNK)
    out = _gather_kernel(idx, word_em)
    return out.reshape(B, L, D)
